# baseline (speedup 1.0000x reference)
import jax
import jax.numpy as jnp
from jax.experimental import pallas as pl
from jax.experimental.pallas import tpu as pltpu

_N, _K, _D, _STAGES = 2048, 512, 64, 4
_NB = 1024


def _exact_gather(onehot, hi, lo):
    gh = jnp.dot(onehot, hi, preferred_element_type=jnp.float32,
                 precision=jax.lax.Precision.HIGHEST)
    gl = jnp.dot(onehot, lo, preferred_element_type=jnp.float32,
                 precision=jax.lax.Precision.HIGHEST)
    bits = (gh.astype(jnp.int32) << 16) | gl.astype(jnp.int32)
    return jax.lax.bitcast_convert_type(bits, jnp.float32)


def _exact_loss_pair(current, c1, c2, x2):
    cc = jnp.concatenate([c1, c2], axis=1)
    curx = jnp.concatenate([current, current], axis=1)
    xx = jnp.concatenate([x2, x2], axis=1)
    cand = curx + cc
    diff = cand - xx
    sq = diff * diff
    acc = sq
    for j in range(1, 8):
        acc = acc + pltpu.roll(sq, 2 * _D - 8 * j, 1)
    for w in (4, 2, 1):
        acc = acc + pltpu.roll(acc, 2 * _D - w, 1)
    dd = acc - pltpu.roll(acc, _D, 1)
    return dd[:, 0:1]


def _encoder_kernel(x_ref, effT_ref, hi_ref, lo_ref, nc_ref,
                    enc_ref, cur_ref, loss_ref):
    x2 = x_ref[...]
    current = jnp.zeros_like(x2)
    iota_k = jax.lax.broadcasted_iota(jnp.int32, (_NB, _K), 1)
    enc_cols = []
    for i in range(_STAGES):
        effT = effT_ref[i]
        r2 = x2 - current
        g = jnp.dot(r2, effT, preferred_element_type=jnp.float32,
                    precision=jax.lax.Precision.HIGHEST)
        nc = nc_ref[pl.ds(i, 1), :]
        q = jnp.sum(r2 * r2, axis=1, keepdims=True)
        mm = (q - 2.0 * g) + nc
        loss_ref[:, pl.ds(i * _K, _K)] = mm
        m1 = jnp.min(mm, axis=1, keepdims=True)
        i1 = jnp.min(jnp.where(mm == m1, iota_k, _K),
                     axis=1, keepdims=True)
        mmm = jnp.where(iota_k == i1, jnp.inf, mm)
        m2 = jnp.min(mmm, axis=1, keepdims=True)
        i2 = jnp.min(jnp.where(mmm == m2, iota_k, _K),
                     axis=1, keepdims=True)
        oh = jnp.concatenate([jnp.where(iota_k == i1, 1.0, 0.0),
                              jnp.where(iota_k == i2, 1.0, 0.0)], axis=0)
        c12 = _exact_gather(oh, hi_ref[i], lo_ref[i])
        c1, c2 = c12[:_NB], c12[_NB:]
        d12 = _exact_loss_pair(current, c1, c2, x2)
        pick1 = (d12 < 0.0) | ((d12 == 0.0) & (i1 < i2))
        idxw = jnp.where(pick1, i1, i2)
        cselw = jnp.where(pick1, c1, c2)
        current = current + cselw
        enc_cols.append(idxw)
    enc_ref[...] = jnp.concatenate(enc_cols, axis=1)
    cur_ref[...] = current


def kernel(inputs, codebook, bias):
    eff = jnp.concatenate([(codebook[0] + bias)[None], codebook[1:]], axis=0)
    effT = jnp.swapaxes(eff, 1, 2)
    bits = jax.lax.bitcast_convert_type(eff, jnp.uint32)
    hi = (bits >> 16).astype(jnp.float32)
    lo = (bits & jnp.uint32(0xFFFF)).astype(jnp.float32)
    nc = jnp.sum(eff * eff, axis=2)
    enc, cur, losses = pl.pallas_call(
        _encoder_kernel,
        grid=(_N // _NB,),
        in_specs=[
            pl.BlockSpec((_NB, _D), lambda m: (m, 0)),
            pl.BlockSpec((_STAGES, _D, _K), lambda m: (0, 0, 0)),
            pl.BlockSpec((_STAGES, _K, _D), lambda m: (0, 0, 0)),
            pl.BlockSpec((_STAGES, _K, _D), lambda m: (0, 0, 0)),
            pl.BlockSpec((_STAGES, _K), lambda m: (0, 0)),
        ],
        out_specs=[
            pl.BlockSpec((_NB, _STAGES), lambda m: (m, 0)),
            pl.BlockSpec((_NB, _D), lambda m: (m, 0)),
            pl.BlockSpec((_NB, _STAGES * _K), lambda m: (m, 0)),
        ],
        out_shape=[
            jax.ShapeDtypeStruct((_N, _STAGES), jnp.int32),
            jax.ShapeDtypeStruct((_N, _D), jnp.float32),
            jax.ShapeDtypeStruct((_N, _STAGES * _K), jnp.float32),
        ],
        compiler_params=pltpu.CompilerParams(
            dimension_semantics=("parallel",)),
    )(inputs, effT, hi, lo, nc)
    return enc, cur, losses.reshape(_N, _STAGES, _K)


# speedup vs baseline: 1.7810x; 1.7810x over previous
import jax
import jax.numpy as jnp
from jax.experimental import pallas as pl
from jax.experimental.pallas import tpu as pltpu

_N, _K, _D, _STAGES = 2048, 512, 64, 4
_NB = 512


def _exact_gather(onehot, planes):
    bits = None
    for s, p in zip((24, 16, 8, 0), planes):
        g = jnp.dot(onehot, p, preferred_element_type=jnp.float32)
        gi = g.astype(jnp.int32) << s
        bits = gi if bits is None else bits | gi
    return jax.lax.bitcast_convert_type(bits, jnp.float32)


def _exact_loss_pair(current, c1, c2, x2):
    cc = jnp.concatenate([c1, c2], axis=1)
    curx = jnp.concatenate([current, current], axis=1)
    xx = jnp.concatenate([x2, x2], axis=1)
    cand = curx + cc
    diff = cand - xx
    sq = diff * diff
    acc = sq
    for j in range(1, 8):
        acc = acc + pltpu.roll(sq, 2 * _D - 8 * j, 1)
    for w in (4, 2, 1):
        acc = acc + pltpu.roll(acc, 2 * _D - w, 1)
    dd = acc - pltpu.roll(acc, _D, 1)
    return dd[:, 0:1]


def _encoder_kernel(x_ref, effT_ref, p3_ref, p2_ref, p1_ref, p0_ref, nc_ref,
                    enc_ref, cur_ref, loss_ref):
    x2 = x_ref[...]
    current = jnp.zeros_like(x2)
    iota_k = jax.lax.broadcasted_iota(jnp.int32, (_NB, _K), 1)
    enc_cols = []
    for i in range(_STAGES):
        effT = effT_ref[i]
        r2 = x2 - current
        g = jnp.dot(r2, effT, preferred_element_type=jnp.float32,
                    precision=jax.lax.Precision.HIGHEST)
        nc = nc_ref[pl.ds(i, 1), :]
        q = jnp.sum(r2 * r2, axis=1, keepdims=True)
        mm = (q - 2.0 * g) + nc
        loss_ref[:, pl.ds(i * _K, _K)] = mm
        m1 = jnp.min(mm, axis=1, keepdims=True)
        i1 = jnp.min(jnp.where(mm == m1, iota_k, _K),
                     axis=1, keepdims=True)
        mmm = jnp.where(iota_k == i1, jnp.inf, mm)
        m2 = jnp.min(mmm, axis=1, keepdims=True)
        i2 = jnp.min(jnp.where(mmm == m2, iota_k, _K),
                     axis=1, keepdims=True)
        oh = jnp.concatenate([jnp.where(iota_k == i1, 1.0, 0.0),
                              jnp.where(iota_k == i2, 1.0, 0.0)],
                             axis=0).astype(jnp.bfloat16)
        c12 = _exact_gather(oh, (p3_ref[i], p2_ref[i],
                                 p1_ref[i], p0_ref[i]))
        c1, c2 = c12[:_NB], c12[_NB:]
        d12 = _exact_loss_pair(current, c1, c2, x2)
        pick1 = (d12 < 0.0) | ((d12 == 0.0) & (i1 < i2))
        idxw = jnp.where(pick1, i1, i2)
        cselw = jnp.where(pick1, c1, c2)
        current = current + cselw
        enc_cols.append(idxw)
    enc_ref[...] = jnp.concatenate(enc_cols, axis=1)
    cur_ref[...] = current


def kernel(inputs, codebook, bias):
    eff = jnp.concatenate([(codebook[0] + bias)[None], codebook[1:]], axis=0)
    effT = jnp.swapaxes(eff, 1, 2)
    bits = jax.lax.bitcast_convert_type(eff, jnp.uint32)
    planes = [((bits >> s) & jnp.uint32(0xFF)).astype(jnp.bfloat16)
              for s in (24, 16, 8, 0)]
    nc = jnp.sum(eff * eff, axis=2)
    enc, cur, losses = pl.pallas_call(
        _encoder_kernel,
        grid=(_N // _NB,),
        in_specs=[
            pl.BlockSpec((_NB, _D), lambda m: (m, 0)),
            pl.BlockSpec((_STAGES, _D, _K), lambda m: (0, 0, 0)),
            pl.BlockSpec((_STAGES, _K, _D), lambda m: (0, 0, 0)),
            pl.BlockSpec((_STAGES, _K, _D), lambda m: (0, 0, 0)),
            pl.BlockSpec((_STAGES, _K, _D), lambda m: (0, 0, 0)),
            pl.BlockSpec((_STAGES, _K, _D), lambda m: (0, 0, 0)),
            pl.BlockSpec((_STAGES, _K), lambda m: (0, 0)),
        ],
        out_specs=[
            pl.BlockSpec((_NB, _STAGES), lambda m: (m, 0)),
            pl.BlockSpec((_NB, _D), lambda m: (m, 0)),
            pl.BlockSpec((_NB, _STAGES * _K), lambda m: (m, 0)),
        ],
        out_shape=[
            jax.ShapeDtypeStruct((_N, _STAGES), jnp.int32),
            jax.ShapeDtypeStruct((_N, _D), jnp.float32),
            jax.ShapeDtypeStruct((_N, _STAGES * _K), jnp.float32),
        ],
        compiler_params=pltpu.CompilerParams(
            dimension_semantics=("parallel",)),
    )(inputs, effT, *planes, nc)
    return enc, cur, losses.reshape(_N, _STAGES, _K)
